# baseline (speedup 1.0000x reference)
import functools

import jax
import jax.numpy as jnp
from jax import lax
from jax.experimental import pallas as pl
from jax.experimental.pallas import tpu as pltpu
from jax.experimental.pallas import tpu_sc as plsc

_V = 1000000
_B = 4096
_H = 200
_D = 64
_DP = 128
_NW = 32
_LANES = 16

_NPAN = _V // _DP
_PPW = _NPAN // _NW
_PEXTRA = _NPAN - _PPW * _NW

_BBLK = _B // _NW
_GBUF = 4


def _iota16():
    return lax.iota(jnp.int32, _LANES)


def _sc_transpose(table_t):
    mesh = plsc.VectorSubcoreMesh(core_axis_name="c", subcore_axis_name="s")

    @functools.partial(
        pl.kernel,
        mesh=mesh,
        compiler_params=pltpu.CompilerParams(needs_layout_passes=False),
        out_type=jax.ShapeDtypeStruct((_V, _DP), jnp.float32),
        scratch_types=[
            *[pltpu.VMEM((_D, _DP), jnp.float32) for _ in range(2)],
            *[pltpu.VMEM((_DP, _DP), jnp.float32) for _ in range(2)],
            pltpu.SemaphoreType.DMA,
            pltpu.SemaphoreType.DMA,
            pltpu.SemaphoreType.DMA,
            pltpu.SemaphoreType.DMA,
        ],
    )
    def k(tt_hbm, out_hbm, vin0, vin1, vout0, vout1, r0, r1, w0, w1):
        vins = (vin0, vin1)
        vouts = (vout0, vout1)
        rsems = (r0, r1)
        wsems = (w0, w1)
        wid = lax.axis_index("s") * 2 + lax.axis_index("c")

        def panel(i):
            return (i * _NW + wid) * _DP

        def start_read(i, b):
            pltpu.make_async_copy(
                tt_hbm.at[:, pl.ds(panel(i), _DP)], vins[b], rsems[b]
            ).start()

        def wait_read(i, b):
            pltpu.make_async_copy(
                tt_hbm.at[:, pl.ds(panel(i), _DP)], vins[b], rsems[b]
            ).wait()

        def transpose(b):
            vin = vins[b]
            vout = vouts[b]

            def trow(r, carry):
                for g in range(_D // _LANES):
                    vals = plsc.load_gather(
                        vin,
                        [_iota16() + (g * _LANES), jnp.full((_LANES,), r, jnp.int32)],
                    )
                    vout[r, pl.ds(g * _LANES, _LANES)] = vals
                return carry

            lax.fori_loop(0, _DP, trow, 0)

        def start_write(i, b):
            pltpu.make_async_copy(
                vouts[b], out_hbm.at[pl.ds(panel(i), _DP)], wsems[b]
            ).start()

        def wait_write(i, b):
            pltpu.make_async_copy(
                vouts[b], out_hbm.at[pl.ds(panel(i), _DP)], wsems[b]
            ).wait()

        start_read(0, 0)
        start_read(1, 1)
        for b in range(2):
            wait_read(b, b)
            transpose(b)
            start_write(b, b)
            start_read(b + 2, b)

        def outer(jo, carry):
            for b in range(2):
                i = jo * 2 + b
                wait_read(i, b)
                wait_write(i - 2, b)
                transpose(b)
                start_write(i, b)
                start_read(i + 2, b)
            return carry

        def outer_safe(jo, carry):
            for b in range(2):
                i = jo * 2 + b
                wait_read(i, b)
                wait_write(i - 2, b)
                transpose(b)
                start_write(i, b)
            return carry

        lax.fori_loop(1, _PPW // 2 - 1, outer, 0, unroll=1)
        outer_safe(_PPW // 2 - 1, 0)

        @pl.when(wid < _PEXTRA)
        def _():
            start_read(_PPW, 0)
            wait_read(_PPW, 0)
            wait_write(_PPW - 2, 0)
            transpose(0)
            start_write(_PPW, 0)
            wait_write(_PPW, 0)

        @pl.when(wid >= _PEXTRA)
        def _():
            wait_write(_PPW - 2, 0)
        wait_write(_PPW - 1, 1)

    return k(table_t)


def _tc_tail(tpad, table_t):
    tail_rows = _V - _NPAN * _DP

    def body(_, tt_ref, out_ref):
        t = tt_ref[...]
        out_ref[:, :_D] = t[:, :tail_rows].T

    return pl.pallas_call(
        body,
        grid=(1,),
        in_specs=[
            pl.BlockSpec(memory_space=pl.ANY),
            pl.BlockSpec((_D, _DP), lambda i: (0, _NPAN)),
        ],
        out_specs=pl.BlockSpec((tail_rows, _DP), lambda i: (_NPAN * 2, 0)),
        out_shape=jax.ShapeDtypeStruct((_V, _DP), jnp.float32),
        input_output_aliases={0: 0},
    )(tpad, table_t)


def _sc_gather(x4, tpad):
    mesh = plsc.VectorSubcoreMesh(core_axis_name="c", subcore_axis_name="s")

    @functools.partial(
        pl.kernel,
        mesh=mesh,
        compiler_params=pltpu.CompilerParams(needs_layout_passes=False),
        out_type=jax.ShapeDtypeStruct((_H, _D, _B), jnp.float32),
        scratch_types=[
            pltpu.VMEM((_H, _BBLK), jnp.int32),
            *[pltpu.VMEM((_BBLK, _DP), jnp.float32) for _ in range(_GBUF)],
            *[pltpu.VMEM((_D, _BBLK), jnp.float32) for _ in range(_GBUF)],
            *[pltpu.SemaphoreType.DMA for _ in range(2 * _GBUF)],
        ],
    )
    def k(x_hbm, table_hbm, out_hbm, idx_v, *rest):
        vg = rest[:_GBUF]
        vt = rest[_GBUF : 2 * _GBUF]
        gsems = rest[2 * _GBUF : 3 * _GBUF]
        wsems = rest[3 * _GBUF :]
        wid = lax.axis_index("s") * 2 + lax.axis_index("c")
        bw = wid * _BBLK

        pltpu.sync_copy(x_hbm.at[wid], idx_v)

        def start_gather(h, b):
            pltpu.make_async_copy(
                table_hbm.at[idx_v.at[h]], vg[b], gsems[b]
            ).start()

        def wait_gather(h, b):
            pltpu.make_async_copy(
                table_hbm.at[idx_v.at[h]], vg[b], gsems[b]
            ).wait()

        def transpose(b):
            srcv = vg[b]
            dst = vt[b]

            def trow(c, carry):
                for g in range(_BBLK // _LANES):
                    vals = plsc.load_gather(
                        srcv,
                        [_iota16() + (g * _LANES), jnp.full((_LANES,), c, jnp.int32)],
                    )
                    dst[c, pl.ds(g * _LANES, _LANES)] = vals
                return carry

            lax.fori_loop(0, _D, trow, 0)

        def start_write(h, b):
            pltpu.make_async_copy(
                vt[b], out_hbm.at[h, :, pl.ds(bw, _BBLK)], wsems[b]
            ).start()

        def wait_write(h, b):
            pltpu.make_async_copy(
                vt[b], out_hbm.at[h, :, pl.ds(bw, _BBLK)], wsems[b]
            ).wait()

        for b in range(_GBUF):
            start_gather(b, b)

        for b in range(_GBUF):
            wait_gather(b, b)
            transpose(b)
            start_write(b, b)
            start_gather(b + _GBUF, b)

        def outer(jo, carry):
            for b in range(_GBUF):
                h = jo * _GBUF + b
                wait_gather(h, b)
                wait_write(h - _GBUF, b)
                transpose(b)
                start_write(h, b)
                start_gather(h + _GBUF, b)
            return carry

        lax.fori_loop(1, _H // _GBUF - 1, outer, 0, unroll=1)

        for b in range(_GBUF):
            h = _H - _GBUF + b
            wait_gather(h, b)
            wait_write(h - _GBUF, b)
            transpose(b)
            start_write(h, b)
        for b in range(_GBUF):
            wait_write(_H - _GBUF + b, b)

    return k(x4, tpad)


def _sc_gather_flat(x3, tpad):
    mesh = plsc.VectorSubcoreMesh(core_axis_name="c", subcore_axis_name="s")

    @functools.partial(
        pl.kernel,
        mesh=mesh,
        out_type=jax.ShapeDtypeStruct((_B * _H, _DP), jnp.float32),
        scratch_types=[
            pltpu.VMEM((200, 128), jnp.int32),
            *[pltpu.VMEM((128, _DP), jnp.float32) for _ in range(4)],
            *[pltpu.SemaphoreType.DMA for _ in range(4)],
        ],
    )
    def k(x_hbm, table_hbm, out_hbm, idx_v, *rest):
        rows = rest[:4]
        sems = rest[4:]
        wid = lax.axis_index("s") * 2 + lax.axis_index("c")
        base = wid * 25600
        pltpu.sync_copy(x_hbm.at[wid], idx_v)

        def start_gather(c, b):
            pltpu.make_async_copy(
                table_hbm.at[idx_v.at[c]], rows[b], sems[b]
            ).start()

        def drain(c, b):
            pltpu.make_async_copy(
                table_hbm.at[idx_v.at[c]], rows[b], sems[b]
            ).wait()
            pltpu.sync_copy(rows[b], out_hbm.at[pl.ds(base + c * 128, 128)])

        for b in range(4):
            start_gather(b, b)

        def outer(jo, carry):
            for b in range(4):
                c = jo * 4 + b
                drain(c, b)
                start_gather(c + 4, b)
            return carry

        lax.fori_loop(0, 49, outer, 0)
        for b in range(4):
            drain(196 + b, b)

    return k(x3, tpad)


_TC_COLS = 2048
_TC_GRID = -(-_V // _TC_COLS)


def _transpose_block(tt_ref, out_ref):
    out_ref[:, :_D] = tt_ref[...].T


def _tc_pad_table(table_t):
    return pl.pallas_call(
        _transpose_block,
        grid=(_TC_GRID,),
        in_specs=[pl.BlockSpec((_D, _TC_COLS), lambda i: (0, i))],
        out_specs=pl.BlockSpec((_TC_COLS, _DP), lambda i: (i, 0)),
        out_shape=jax.ShapeDtypeStruct((_V, _DP), jnp.float32),
    )(table_t)

def kernel(x, table):
    tpad = _sc_transpose(table.T)
    tpad = _tc_tail(tpad, table.T)
    out = _sc_gather_flat(x.reshape(_NW, 200, 128), tpad)
    return out[:, :_D].reshape(_B, _H, _D)


# speedup vs baseline: 2.6255x; 2.6255x over previous
import functools

import jax
import jax.numpy as jnp
from jax import lax
from jax.experimental import pallas as pl
from jax.experimental.pallas import tpu as pltpu
from jax.experimental.pallas import tpu_sc as plsc

_V = 1000000
_B = 4096
_H = 200
_D = 64
_DP = 128
_NW = 32
_TOT = _B * _H
_RPW = _TOT // _NW
_CH = 128
_NCH = _RPW // _CH
_NBUF = 4

_TC_COLS = 8192
_TC_GRID = -(-_V // _TC_COLS)


def _transpose_block(tt_ref, out_ref):
    out_ref[:, :_D] = tt_ref[...].T


def _pad_table(table_t):
    return pl.pallas_call(
        _transpose_block,
        grid=(_TC_GRID,),
        in_specs=[pl.BlockSpec((_D, _TC_COLS), lambda i: (0, i))],
        out_specs=pl.BlockSpec((_TC_COLS, _DP), lambda i: (i, 0)),
        out_shape=jax.ShapeDtypeStruct((_V, _DP), jnp.float32),
    )(table_t)


def _sc_gather(x3, tpad):
    mesh = plsc.VectorSubcoreMesh(core_axis_name="c", subcore_axis_name="s")

    @functools.partial(
        pl.kernel,
        mesh=mesh,
        out_type=jax.ShapeDtypeStruct((_TOT, _DP), jnp.float32),
        scratch_types=[
            pltpu.VMEM((_NCH, _CH), jnp.int32),
            *[pltpu.VMEM((_CH, _DP), jnp.float32) for _ in range(_NBUF)],
            *[pltpu.SemaphoreType.DMA for _ in range(_NBUF)],
        ],
    )
    def k(x_hbm, table_hbm, out_hbm, idx_v, *rest):
        rows = rest[:_NBUF]
        sems = rest[_NBUF:]
        wid = lax.axis_index("s") * 2 + lax.axis_index("c")
        base = wid * _RPW

        pltpu.sync_copy(x_hbm.at[wid], idx_v)

        def start_gather(c, b):
            pltpu.make_async_copy(
                table_hbm.at[idx_v.at[c]], rows[b], sems[b]
            ).start()

        def drain(c, b):
            pltpu.make_async_copy(
                table_hbm.at[idx_v.at[c]], rows[b], sems[b]
            ).wait()
            pltpu.sync_copy(rows[b], out_hbm.at[pl.ds(base + c * _CH, _CH)])

        for b in range(_NBUF):
            start_gather(b, b)

        def outer(jo, carry):
            for b in range(_NBUF):
                c = jo * _NBUF + b
                drain(c, b)
                start_gather(c + _NBUF, b)
            return carry

        lax.fori_loop(0, _NCH // _NBUF - 1, outer, 0)

        for b in range(_NBUF):
            drain(_NCH - _NBUF + b, b)

    return k(x3, tpad)


def kernel(x, table):
    x3 = x.reshape(_NW, _NCH, _CH)
    tpad = _pad_table(table.T)
    out = _sc_gather(x3, tpad)
    return out[:, :_D].reshape(_B, _H, _D)


# speedup vs baseline: 2.6924x; 1.0255x over previous
import functools

import jax
import jax.numpy as jnp
from jax import lax
from jax.experimental import pallas as pl
from jax.experimental.pallas import tpu as pltpu
from jax.experimental.pallas import tpu_sc as plsc

_V = 1000000
_B = 4096
_H = 200
_D = 64
_DP = 128
_NW = 32
_TOT = _B * _H
_RPW = _TOT // _NW
_CH = 128
_NCH = _RPW // _CH
_NBUF = 4

_TC_COLS = 16384
_TC_GRID = -(-_V // _TC_COLS)


def _transpose_block(tt_ref, out_ref):
    out_ref[:, :_D] = tt_ref[...].T


def _pad_table(table_t):
    return pl.pallas_call(
        _transpose_block,
        grid=(_TC_GRID,),
        in_specs=[pl.BlockSpec((_D, _TC_COLS), lambda i: (0, i))],
        out_specs=pl.BlockSpec((_TC_COLS, _DP), lambda i: (i, 0)),
        out_shape=jax.ShapeDtypeStruct((_V, _DP), jnp.float32),
    )(table_t)


def _sc_gather(x3, tpad):
    mesh = plsc.VectorSubcoreMesh(core_axis_name="c", subcore_axis_name="s")

    @functools.partial(
        pl.kernel,
        mesh=mesh,
        out_type=jax.ShapeDtypeStruct((_TOT, _DP), jnp.float32),
        scratch_types=[
            pltpu.VMEM((_NCH, _CH), jnp.int32),
            *[pltpu.VMEM((_CH, _DP), jnp.float32) for _ in range(_NBUF)],
            *[pltpu.SemaphoreType.DMA for _ in range(_NBUF)],
        ],
    )
    def k(x_hbm, table_hbm, out_hbm, idx_v, *rest):
        rows = rest[:_NBUF]
        sems = rest[_NBUF:]
        wid = lax.axis_index("s") * 2 + lax.axis_index("c")
        base = wid * _RPW

        pltpu.sync_copy(x_hbm.at[wid], idx_v)

        def start_gather(c, b):
            pltpu.make_async_copy(
                table_hbm.at[idx_v.at[c]], rows[b], sems[b]
            ).start()

        def drain(c, b):
            pltpu.make_async_copy(
                table_hbm.at[idx_v.at[c]], rows[b], sems[b]
            ).wait()
            pltpu.sync_copy(rows[b], out_hbm.at[pl.ds(base + c * _CH, _CH)])

        for b in range(_NBUF):
            start_gather(b, b)

        def outer(jo, carry):
            for b in range(_NBUF):
                c = jo * _NBUF + b
                drain(c, b)
                start_gather(c + _NBUF, b)
            return carry

        lax.fori_loop(0, _NCH // _NBUF - 1, outer, 0)

        for b in range(_NBUF):
            drain(_NCH - _NBUF + b, b)

    return k(x3, tpad)


def kernel(x, table):
    x3 = x.reshape(_NW, _NCH, _CH)
    tpad = _pad_table(table.T)
    out = _sc_gather(x3, tpad)
    return out[:, :_D].reshape(_B, _H, _D)


# speedup vs baseline: 2.7111x; 1.0069x over previous
import functools

import jax
import jax.numpy as jnp
from jax import lax
from jax.experimental import pallas as pl
from jax.experimental.pallas import tpu as pltpu
from jax.experimental.pallas import tpu_sc as plsc

_V = 1000000
_B = 4096
_H = 200
_D = 64
_DP = 128
_NW = 32
_TOT = _B * _H
_RPW = _TOT // _NW
_CH = 128
_NCH = _RPW // _CH
_NBUF = 4

_TC_COLS = 32768
_TC_GRID = -(-_V // _TC_COLS)


def _transpose_block(tt_ref, out_ref):
    out_ref[:, :_D] = tt_ref[...].T


def _pad_table(table_t):
    return pl.pallas_call(
        _transpose_block,
        grid=(_TC_GRID,),
        in_specs=[pl.BlockSpec((_D, _TC_COLS), lambda i: (0, i))],
        out_specs=pl.BlockSpec((_TC_COLS, _DP), lambda i: (i, 0)),
        out_shape=jax.ShapeDtypeStruct((_V, _DP), jnp.float32),
    )(table_t)


def _sc_gather(x3, tpad):
    mesh = plsc.VectorSubcoreMesh(core_axis_name="c", subcore_axis_name="s")

    @functools.partial(
        pl.kernel,
        mesh=mesh,
        out_type=jax.ShapeDtypeStruct((_TOT, _DP), jnp.float32),
        scratch_types=[
            pltpu.VMEM((_NCH, _CH), jnp.int32),
            *[pltpu.VMEM((_CH, _DP), jnp.float32) for _ in range(_NBUF)],
            *[pltpu.SemaphoreType.DMA for _ in range(_NBUF)],
        ],
    )
    def k(x_hbm, table_hbm, out_hbm, idx_v, *rest):
        rows = rest[:_NBUF]
        sems = rest[_NBUF:]
        wid = lax.axis_index("s") * 2 + lax.axis_index("c")
        base = wid * _RPW

        pltpu.sync_copy(x_hbm.at[wid], idx_v)

        def start_gather(c, b):
            pltpu.make_async_copy(
                table_hbm.at[idx_v.at[c]], rows[b], sems[b]
            ).start()

        def drain(c, b):
            pltpu.make_async_copy(
                table_hbm.at[idx_v.at[c]], rows[b], sems[b]
            ).wait()
            pltpu.sync_copy(rows[b], out_hbm.at[pl.ds(base + c * _CH, _CH)])

        for b in range(_NBUF):
            start_gather(b, b)

        def outer(jo, carry):
            for b in range(_NBUF):
                c = jo * _NBUF + b
                drain(c, b)
                start_gather(c + _NBUF, b)
            return carry

        lax.fori_loop(0, _NCH // _NBUF - 1, outer, 0)

        for b in range(_NBUF):
            drain(_NCH - _NBUF + b, b)

    return k(x3, tpad)


def kernel(x, table):
    x3 = x.reshape(_NW, _NCH, _CH)
    tpad = _pad_table(table.T)
    out = _sc_gather(x3, tpad)
    return out[:, :_D].reshape(_B, _H, _D)


# speedup vs baseline: 2.7129x; 1.0007x over previous
import functools

import jax
import jax.numpy as jnp
from jax import lax
from jax.experimental import pallas as pl
from jax.experimental.pallas import tpu as pltpu
from jax.experimental.pallas import tpu_sc as plsc

_V = 1000000
_B = 4096
_H = 200
_D = 64
_DP = 128
_NW = 32
_TOT = _B * _H
_RPW = _TOT // _NW
_CH = 128
_NCH = _RPW // _CH
_NBUF = 5

_TC_COLS = 32768
_TC_GRID = -(-_V // _TC_COLS)


def _transpose_block(tt_ref, out_ref):
    out_ref[:, :_D] = tt_ref[...].T


def _pad_table(table_t):
    return pl.pallas_call(
        _transpose_block,
        grid=(_TC_GRID,),
        in_specs=[pl.BlockSpec((_D, _TC_COLS), lambda i: (0, i))],
        out_specs=pl.BlockSpec((_TC_COLS, _DP), lambda i: (i, 0)),
        out_shape=jax.ShapeDtypeStruct((_V, _DP), jnp.float32),
    )(table_t)


def _sc_gather(x3, tpad):
    mesh = plsc.VectorSubcoreMesh(core_axis_name="c", subcore_axis_name="s")

    @functools.partial(
        pl.kernel,
        mesh=mesh,
        out_type=jax.ShapeDtypeStruct((_TOT, _DP), jnp.float32),
        scratch_types=[
            pltpu.VMEM((_NCH, _CH), jnp.int32),
            *[pltpu.VMEM((_CH, _DP), jnp.float32) for _ in range(_NBUF)],
            *[pltpu.SemaphoreType.DMA for _ in range(_NBUF)],
        ],
    )
    def k(x_hbm, table_hbm, out_hbm, idx_v, *rest):
        rows = rest[:_NBUF]
        sems = rest[_NBUF:]
        wid = lax.axis_index("s") * 2 + lax.axis_index("c")
        base = wid * _RPW

        pltpu.sync_copy(x_hbm.at[wid], idx_v)

        def start_gather(c, b):
            pltpu.make_async_copy(
                table_hbm.at[idx_v.at[c]], rows[b], sems[b]
            ).start()

        def drain(c, b):
            pltpu.make_async_copy(
                table_hbm.at[idx_v.at[c]], rows[b], sems[b]
            ).wait()
            pltpu.sync_copy(rows[b], out_hbm.at[pl.ds(base + c * _CH, _CH)])

        for b in range(_NBUF):
            start_gather(b, b)

        def outer(jo, carry):
            for b in range(_NBUF):
                c = jo * _NBUF + b
                drain(c, b)
                start_gather(c + _NBUF, b)
            return carry

        lax.fori_loop(0, _NCH // _NBUF - 1, outer, 0)

        for b in range(_NBUF):
            drain(_NCH - _NBUF + b, b)

    return k(x3, tpad)


def kernel(x, table):
    x3 = x.reshape(_NW, _NCH, _CH)
    tpad = _pad_table(table.T)
    out = _sc_gather(x3, tpad)
    return out[:, :_D].reshape(_B, _H, _D)
